# baseline (device time: 521575 ns/iter reference)
import jax
import jax.numpy as jnp
from jax import lax
from jax.experimental import pallas as pl
from jax.experimental.pallas import tpu as pltpu

N_DEV = 4
KT = 1024


def kernel(x, w_mat):
    m_per, k_dim = x.shape
    n_per = w_mat.shape[1]
    nk = k_dim // KT
    khalf = k_dim // 2

    def body(x_ref, w_ref, out_ref, wfull,
             xbuf, wbuf, acc, bsend, brecv,
             wsend_sems, wrecv_sems, bsend_sems, brecv_sems,
             copy_sems, local_sem):
        me = lax.axis_index("i")
        left = (me - 1) % N_DEV
        right = (me + 1) % N_DEV
        opp = (me + 2) % N_DEV

        barrier_sem = pltpu.get_barrier_semaphore()
        for nbr in [left, right, opp]:
            pl.semaphore_signal(
                barrier_sem, inc=1,
                device_id=(nbr,), device_id_type=pl.DeviceIdType.MESH,
            )
        pl.semaphore_wait(barrier_sem, 3)

        def w_rdma(src_slice, dst_slot, dst_slice, dev, si):
            return pltpu.make_async_remote_copy(
                src_ref=src_slice, dst_ref=wfull.at[dst_slot, dst_slice],
                send_sem=wsend_sems.at[si], recv_sem=wrecv_sems.at[si],
                device_id=(dev,), device_id_type=pl.DeviceIdType.MESH)

        top = pl.ds(0, khalf)
        bot = pl.ds(khalf, khalf)
        cw_top = w_rdma(w_ref.at[top], me, top, right, 0)
        ccw_bot = w_rdma(w_ref.at[bot], me, bot, left, 1)
        cw_top.start()
        ccw_bot.start()

        def compute_block(wsrc_ref):
            def mk_copies(ki, buf):
                cx = pltpu.make_async_copy(
                    x_ref.at[:, pl.ds(ki * KT, KT)],
                    xbuf.at[buf], copy_sems.at[buf])
                cw = pltpu.make_async_copy(
                    wsrc_ref.at[pl.ds(ki * KT, KT), :],
                    wbuf.at[buf], copy_sems.at[2 + buf])
                return cx, cw

            def start_copies(ki, buf):
                cx, cw = mk_copies(ki, buf)
                cx.start()
                cw.start()

            start_copies(0, 0)

            def step(ki, carry):
                buf = lax.rem(ki, 2)

                @pl.when(ki + 1 < nk)
                def _():
                    start_copies(ki + 1, 1 - buf)

                cx, cw = mk_copies(ki, buf)
                cx.wait()
                cw.wait()
                p = jnp.dot(xbuf[buf], wbuf[buf],
                            preferred_element_type=jnp.float32)

                @pl.when(ki == 0)
                def _():
                    acc[...] = p

                @pl.when(ki > 0)
                def _():
                    acc[...] += p

                return carry

            lax.fori_loop(0, nk, step, 0)
            y = acc[...]
            c = 0.7978845608028654
            acc[...] = 0.5 * y * (1.0 + jnp.tanh(c * (y + 0.044715 * y * y * y)))

        compute_block(w_ref)
        own_cp = pltpu.make_async_copy(
            acc, out_ref.at[pl.ds(me * m_per, m_per), :], local_sem)
        own_cp.start()
        own_cp.wait()

        cw_top.wait_recv()
        fwd_cw = w_rdma(wfull.at[left, top], left, top, right, 2)
        fwd_cw.start()
        ccw_bot.wait_recv()
        fwd_ccw = w_rdma(wfull.at[right, bot], right, bot, left, 3)
        fwd_ccw.start()

        fwd_cw.wait_send()
        cw_bot = w_rdma(w_ref.at[bot], me, bot, right, 4)
        cw_bot.start()
        fwd_ccw.wait_send()
        ccw_top = w_rdma(w_ref.at[top], me, top, left, 5)
        ccw_top.start()

        fwd_cw.wait_recv()
        fwd_ccw.wait_recv()

        compute_block(wfull.at[opp])
        bsend[2] = acc[...].astype(jnp.bfloat16)
        b_o = pltpu.make_async_remote_copy(
            src_ref=bsend.at[2], dst_ref=brecv.at[2],
            send_sem=bsend_sems.at[2], recv_sem=brecv_sems.at[2],
            device_id=(opp,), device_id_type=pl.DeviceIdType.MESH)
        b_o.start()

        cw_bot.wait_recv()
        compute_block(wfull.at[left])
        bsend[1] = acc[...].astype(jnp.bfloat16)
        b_l = pltpu.make_async_remote_copy(
            src_ref=bsend.at[1], dst_ref=brecv.at[1],
            send_sem=bsend_sems.at[1], recv_sem=brecv_sems.at[1],
            device_id=(left,), device_id_type=pl.DeviceIdType.MESH)
        b_l.start()

        ccw_top.wait_recv()
        compute_block(wfull.at[right])
        bsend[0] = acc[...].astype(jnp.bfloat16)
        b_r = pltpu.make_async_remote_copy(
            src_ref=bsend.at[0], dst_ref=brecv.at[0],
            send_sem=bsend_sems.at[0], recv_sem=brecv_sems.at[0],
            device_id=(right,), device_id_type=pl.DeviceIdType.MESH)
        b_r.start()

        for s, origin in [(0, left), (1, right), (2, opp)]:
            rwait = pltpu.make_async_remote_copy(
                src_ref=bsend.at[s], dst_ref=brecv.at[s],
                send_sem=bsend_sems.at[s], recv_sem=brecv_sems.at[s],
                device_id=(me,), device_id_type=pl.DeviceIdType.MESH)
            rwait.wait_recv()
            acc[...] = brecv[s].astype(jnp.float32)
            st = pltpu.make_async_copy(
                acc, out_ref.at[pl.ds(origin * m_per, m_per), :], local_sem)
            st.start()
            st.wait()

        cw_top.wait_send()
        ccw_bot.wait_send()
        cw_bot.wait_send()
        ccw_top.wait_send()
        b_r.wait_send()
        b_l.wait_send()
        b_o.wait_send()

    out, _wfull = pl.pallas_call(
        body,
        out_shape=[
            jax.ShapeDtypeStruct((N_DEV * m_per, n_per), jnp.float32),
            jax.ShapeDtypeStruct((N_DEV, k_dim, n_per), jnp.bfloat16),
        ],
        in_specs=[
            pl.BlockSpec(memory_space=pl.ANY),
            pl.BlockSpec(memory_space=pl.ANY),
        ],
        out_specs=[
            pl.BlockSpec(memory_space=pl.ANY),
            pl.BlockSpec(memory_space=pl.ANY),
        ],
        scratch_shapes=[
            pltpu.VMEM((2, m_per, KT), jnp.bfloat16),
            pltpu.VMEM((2, KT, n_per), jnp.bfloat16),
            pltpu.VMEM((m_per, n_per), jnp.float32),
            pltpu.VMEM((3, m_per, n_per), jnp.bfloat16),
            pltpu.VMEM((3, m_per, n_per), jnp.bfloat16),
            pltpu.SemaphoreType.DMA((6,)),
            pltpu.SemaphoreType.DMA((6,)),
            pltpu.SemaphoreType.DMA((3,)),
            pltpu.SemaphoreType.DMA((3,)),
            pltpu.SemaphoreType.DMA((4,)),
            pltpu.SemaphoreType.DMA,
        ],
        compiler_params=pltpu.CompilerParams(
            collective_id=0,
            vmem_limit_bytes=60 * 1024 * 1024,
        ),
    )(x.astype(jnp.bfloat16), w_mat.astype(jnp.bfloat16))
    return out


# device time: 431477 ns/iter; 1.2088x vs baseline; 1.2088x over previous
import jax
import jax.numpy as jnp
from jax import lax
from jax.experimental import pallas as pl
from jax.experimental.pallas import tpu as pltpu

N_DEV = 4
KT = 1024


def kernel(x, w_mat):
    m_per, k_dim = x.shape
    n_per = w_mat.shape[1]
    nk = k_dim // KT
    khalf = k_dim // 2

    def body(x_ref, w_ref, out_ref, wfull,
             xbuf, wbuf, acc, bsend, brecv,
             wsend_sems, wrecv_sems, bsend_sems, brecv_sems,
             copy_sems, local_sem):
        me = lax.axis_index("i")
        left = (me - 1) % N_DEV
        right = (me + 1) % N_DEV
        opp = (me + 2) % N_DEV

        barrier_sem = pltpu.get_barrier_semaphore()
        for nbr in [left, right, opp]:
            pl.semaphore_signal(
                barrier_sem, inc=1,
                device_id=(nbr,), device_id_type=pl.DeviceIdType.MESH,
            )
        pl.semaphore_wait(barrier_sem, 3)

        wr0 = pltpu.make_async_remote_copy(
            src_ref=w_ref, dst_ref=wfull.at[me],
            send_sem=wsend_sems.at[0], recv_sem=wrecv_sems.at[0],
            device_id=(right,), device_id_type=pl.DeviceIdType.MESH)
        wl0 = pltpu.make_async_remote_copy(
            src_ref=w_ref, dst_ref=wfull.at[me],
            send_sem=wsend_sems.at[1], recv_sem=wrecv_sems.at[1],
            device_id=(left,), device_id_type=pl.DeviceIdType.MESH)
        wr0.start()
        wl0.start()

        def compute_block(wsrc_ref):
            def mk_copies(ki, buf):
                cx = pltpu.make_async_copy(
                    x_ref.at[:, pl.ds(ki * KT, KT)],
                    xbuf.at[buf], copy_sems.at[buf])
                cw = pltpu.make_async_copy(
                    wsrc_ref.at[pl.ds(ki * KT, KT), :],
                    wbuf.at[buf], copy_sems.at[2 + buf])
                return cx, cw

            def start_copies(ki, buf):
                cx, cw = mk_copies(ki, buf)
                cx.start()
                cw.start()

            start_copies(0, 0)

            def step(ki, carry):
                buf = lax.rem(ki, 2)

                @pl.when(ki + 1 < nk)
                def _():
                    start_copies(ki + 1, 1 - buf)

                cx, cw = mk_copies(ki, buf)
                cx.wait()
                cw.wait()
                p = jnp.dot(xbuf[buf].astype(jnp.bfloat16), wbuf[buf],
                            preferred_element_type=jnp.float32)

                @pl.when(ki == 0)
                def _():
                    acc[...] = p

                @pl.when(ki > 0)
                def _():
                    acc[...] += p

                return carry

            lax.fori_loop(0, nk, step, 0)
            y = acc[...]
            c = 0.7978845608028654
            acc[...] = 0.5 * y * (1.0 + jnp.tanh(c * (y + 0.044715 * y * y * y)))

        compute_block(w_ref)
        own_cp = pltpu.make_async_copy(
            acc, out_ref.at[pl.ds(me * m_per, m_per), :], local_sem)
        own_cp.start()
        own_cp.wait()

        wr0.wait_recv()
        wl0.wait_recv()

        wr1 = pltpu.make_async_remote_copy(
            src_ref=wfull.at[left, pl.ds(0, khalf)],
            dst_ref=wfull.at[left, pl.ds(0, khalf)],
            send_sem=wsend_sems.at[2], recv_sem=wrecv_sems.at[2],
            device_id=(right,), device_id_type=pl.DeviceIdType.MESH)
        wl1 = pltpu.make_async_remote_copy(
            src_ref=wfull.at[right, pl.ds(khalf, khalf)],
            dst_ref=wfull.at[right, pl.ds(khalf, khalf)],
            send_sem=wsend_sems.at[3], recv_sem=wrecv_sems.at[3],
            device_id=(left,), device_id_type=pl.DeviceIdType.MESH)
        wr1.start()
        wl1.start()

        compute_block(wfull.at[right])
        bsend[0] = acc[...].astype(jnp.bfloat16)
        b_r = pltpu.make_async_remote_copy(
            src_ref=bsend.at[0], dst_ref=brecv.at[0],
            send_sem=bsend_sems.at[0], recv_sem=brecv_sems.at[0],
            device_id=(right,), device_id_type=pl.DeviceIdType.MESH)
        b_r.start()

        compute_block(wfull.at[left])
        bsend[1] = acc[...].astype(jnp.bfloat16)
        b_l = pltpu.make_async_remote_copy(
            src_ref=bsend.at[1], dst_ref=brecv.at[1],
            send_sem=bsend_sems.at[1], recv_sem=brecv_sems.at[1],
            device_id=(left,), device_id_type=pl.DeviceIdType.MESH)
        b_l.start()

        wr1.wait_recv()
        wl1.wait_recv()

        compute_block(wfull.at[opp])
        bsend[2] = acc[...].astype(jnp.bfloat16)
        b_o = pltpu.make_async_remote_copy(
            src_ref=bsend.at[2], dst_ref=brecv.at[2],
            send_sem=bsend_sems.at[2], recv_sem=brecv_sems.at[2],
            device_id=(opp,), device_id_type=pl.DeviceIdType.MESH)
        b_o.start()

        for s, origin in [(0, left), (1, right), (2, opp)]:
            rwait = pltpu.make_async_remote_copy(
                src_ref=bsend.at[s], dst_ref=brecv.at[s],
                send_sem=bsend_sems.at[s], recv_sem=brecv_sems.at[s],
                device_id=(me,), device_id_type=pl.DeviceIdType.MESH)
            rwait.wait_recv()
            acc[...] = brecv[s].astype(jnp.float32)
            st = pltpu.make_async_copy(
                acc, out_ref.at[pl.ds(origin * m_per, m_per), :], local_sem)
            st.start()
            st.wait()

        wr0.wait_send()
        wl0.wait_send()
        wr1.wait_send()
        wl1.wait_send()
        b_r.wait_send()
        b_l.wait_send()
        b_o.wait_send()

    out, _wfull = pl.pallas_call(
        body,
        out_shape=[
            jax.ShapeDtypeStruct((N_DEV * m_per, n_per), jnp.float32),
            jax.ShapeDtypeStruct((N_DEV, k_dim, n_per), jnp.bfloat16),
        ],
        in_specs=[
            pl.BlockSpec(memory_space=pl.ANY),
            pl.BlockSpec(memory_space=pl.ANY),
        ],
        out_specs=[
            pl.BlockSpec(memory_space=pl.ANY),
            pl.BlockSpec(memory_space=pl.ANY),
        ],
        scratch_shapes=[
            pltpu.VMEM((2, m_per, KT), jnp.float32),
            pltpu.VMEM((2, KT, n_per), jnp.bfloat16),
            pltpu.VMEM((m_per, n_per), jnp.float32),
            pltpu.VMEM((3, m_per, n_per), jnp.bfloat16),
            pltpu.VMEM((3, m_per, n_per), jnp.bfloat16),
            pltpu.SemaphoreType.DMA((4,)),
            pltpu.SemaphoreType.DMA((4,)),
            pltpu.SemaphoreType.DMA((3,)),
            pltpu.SemaphoreType.DMA((3,)),
            pltpu.SemaphoreType.DMA((4,)),
            pltpu.SemaphoreType.DMA,
        ],
        compiler_params=pltpu.CompilerParams(
            collective_id=0,
            vmem_limit_bytes=63 * 1024 * 1024,
        ),
    )(x, w_mat.astype(jnp.bfloat16))
    return out


# device time: 423554 ns/iter; 1.2314x vs baseline; 1.0187x over previous
import jax
import jax.numpy as jnp
from jax import lax
from jax.experimental import pallas as pl
from jax.experimental.pallas import tpu as pltpu

N_DEV = 4
KT = 1024


def kernel(x, w_mat):
    m_per, k_dim = x.shape
    n_per = w_mat.shape[1]
    nk = k_dim // KT
    khalf = k_dim // 2

    def body(x_ref, w_ref, out_ref, wfull,
             xbuf, wbuf, acc, bsend, brecv,
             wsend_sems, wrecv_sems, bsend_sems, brecv_sems,
             copy_sems, local_sem):
        me = lax.axis_index("i")
        left = (me - 1) % N_DEV
        right = (me + 1) % N_DEV
        opp = (me + 2) % N_DEV

        barrier_sem = pltpu.get_barrier_semaphore()
        for nbr in [left, right, opp]:
            pl.semaphore_signal(
                barrier_sem, inc=1,
                device_id=(nbr,), device_id_type=pl.DeviceIdType.MESH,
            )
        pl.semaphore_wait(barrier_sem, 3)

        top = pl.ds(0, khalf)
        bot = pl.ds(khalf, khalf)

        def w_rdma(src, dst, dev, si):
            return pltpu.make_async_remote_copy(
                src_ref=src, dst_ref=dst,
                send_sem=wsend_sems.at[si], recv_sem=wrecv_sems.at[si],
                device_id=(dev,), device_id_type=pl.DeviceIdType.MESH)

        cw_a = w_rdma(w_ref.at[top], wfull.at[me, top], right, 0)
        ccw_a = w_rdma(w_ref.at[top], wfull.at[me, top], left, 1)
        cw_a.start()
        ccw_a.start()

        def compute_part(wsrc_ref, k0, ntiles, first):
            def mk_copies(ki, buf):
                cx = pltpu.make_async_copy(
                    x_ref.at[:, pl.ds(ki * KT, KT)],
                    xbuf.at[buf], copy_sems.at[buf])
                cw = pltpu.make_async_copy(
                    wsrc_ref.at[pl.ds(ki * KT, KT), :],
                    wbuf.at[buf], copy_sems.at[2 + buf])
                return cx, cw

            def start_copies(ki, buf):
                cx, cw = mk_copies(ki, buf)
                cx.start()
                cw.start()

            start_copies(k0, 0)

            def step(i, carry):
                ki = k0 + i
                buf = lax.rem(i, 2)

                @pl.when(i + 1 < ntiles)
                def _():
                    start_copies(ki + 1, 1 - buf)

                cx, cw = mk_copies(ki, buf)
                cx.wait()
                cw.wait()
                p = jnp.dot(xbuf[buf].astype(jnp.bfloat16), wbuf[buf],
                            preferred_element_type=jnp.float32)

                if first:
                    @pl.when(i == 0)
                    def _():
                        acc[...] = p

                    @pl.when(i > 0)
                    def _():
                        acc[...] += p
                else:
                    acc[...] += p

                return carry

            lax.fori_loop(0, ntiles, step, 0)

        def finish_gelu():
            y = acc[...]
            c = 0.7978845608028654
            acc[...] = 0.5 * y * (1.0 + jnp.tanh(c * (y + 0.044715 * y * y * y)))

        def compute_block(wsrc_ref):
            compute_part(wsrc_ref, 0, nk, first=True)
            finish_gelu()

        compute_block(w_ref)
        own_cp = pltpu.make_async_copy(
            acc, out_ref.at[pl.ds(me * m_per, m_per), :], local_sem)
        own_cp.start()
        own_cp.wait()

        cw_a.wait_send()
        cw_b = w_rdma(w_ref.at[bot], wfull.at[me, bot], right, 4)
        cw_b.start()
        ccw_a.wait_send()
        ccw_b = w_rdma(w_ref.at[bot], wfull.at[me, bot], left, 5)
        ccw_b.start()

        cw_a.wait_recv()
        ccw_a.wait_recv()

        compute_part(wfull.at[right], 0, nk // 2, first=True)

        cw_b.wait_send()
        fwd_cw = w_rdma(wfull.at[left, top], wfull.at[left, top], right, 2)
        fwd_cw.start()
        ccw_b.wait_send()
        ccw_b.wait_recv()
        fwd_ccw = w_rdma(wfull.at[right, bot], wfull.at[right, bot], left, 3)
        fwd_ccw.start()

        compute_part(wfull.at[right], nk // 2, nk // 2, first=False)
        finish_gelu()
        bsend[0] = acc[...].astype(jnp.bfloat16)
        b_r = pltpu.make_async_remote_copy(
            src_ref=bsend.at[0], dst_ref=brecv.at[0],
            send_sem=bsend_sems.at[0], recv_sem=brecv_sems.at[0],
            device_id=(right,), device_id_type=pl.DeviceIdType.MESH)
        b_r.start()

        cw_b.wait_recv()
        compute_block(wfull.at[left])
        bsend[1] = acc[...].astype(jnp.bfloat16)
        b_l = pltpu.make_async_remote_copy(
            src_ref=bsend.at[1], dst_ref=brecv.at[1],
            send_sem=bsend_sems.at[1], recv_sem=brecv_sems.at[1],
            device_id=(left,), device_id_type=pl.DeviceIdType.MESH)
        b_l.start()

        fwd_cw.wait_recv()
        fwd_ccw.wait_recv()

        compute_block(wfull.at[opp])
        bsend[2] = acc[...].astype(jnp.bfloat16)
        b_o = pltpu.make_async_remote_copy(
            src_ref=bsend.at[2], dst_ref=brecv.at[2],
            send_sem=bsend_sems.at[2], recv_sem=brecv_sems.at[2],
            device_id=(opp,), device_id_type=pl.DeviceIdType.MESH)
        b_o.start()

        for s, origin in [(0, left), (1, right), (2, opp)]:
            rwait = pltpu.make_async_remote_copy(
                src_ref=bsend.at[s], dst_ref=brecv.at[s],
                send_sem=bsend_sems.at[s], recv_sem=brecv_sems.at[s],
                device_id=(me,), device_id_type=pl.DeviceIdType.MESH)
            rwait.wait_recv()
            acc[...] = brecv[s].astype(jnp.float32)
            st = pltpu.make_async_copy(
                acc, out_ref.at[pl.ds(origin * m_per, m_per), :], local_sem)
            st.start()
            st.wait()

        fwd_cw.wait_send()
        fwd_ccw.wait_send()
        b_r.wait_send()
        b_l.wait_send()
        b_o.wait_send()

    out, _wfull = pl.pallas_call(
        body,
        out_shape=[
            jax.ShapeDtypeStruct((N_DEV * m_per, n_per), jnp.float32),
            jax.ShapeDtypeStruct((N_DEV, k_dim, n_per), jnp.bfloat16),
        ],
        in_specs=[
            pl.BlockSpec(memory_space=pl.ANY),
            pl.BlockSpec(memory_space=pl.ANY),
        ],
        out_specs=[
            pl.BlockSpec(memory_space=pl.ANY),
            pl.BlockSpec(memory_space=pl.ANY),
        ],
        scratch_shapes=[
            pltpu.VMEM((2, m_per, KT), jnp.float32),
            pltpu.VMEM((2, KT, n_per), jnp.bfloat16),
            pltpu.VMEM((m_per, n_per), jnp.float32),
            pltpu.VMEM((3, m_per, n_per), jnp.bfloat16),
            pltpu.VMEM((3, m_per, n_per), jnp.bfloat16),
            pltpu.SemaphoreType.DMA((6,)),
            pltpu.SemaphoreType.DMA((6,)),
            pltpu.SemaphoreType.DMA((3,)),
            pltpu.SemaphoreType.DMA((3,)),
            pltpu.SemaphoreType.DMA((4,)),
            pltpu.SemaphoreType.DMA,
        ],
        compiler_params=pltpu.CompilerParams(
            collective_id=0,
            vmem_limit_bytes=63 * 1024 * 1024,
        ),
    )(x, w_mat.astype(jnp.bfloat16))
    return out


# device time: 421971 ns/iter; 1.2360x vs baseline; 1.0038x over previous
import jax
import jax.numpy as jnp
from jax import lax
from jax.experimental import pallas as pl
from jax.experimental.pallas import tpu as pltpu

N_DEV = 4
KT = 1024


def kernel(x, w_mat):
    m_per, k_dim = x.shape
    n_per = w_mat.shape[1]
    nk = k_dim // KT
    khalf = k_dim // 2

    def body(x_ref, w_ref, out_ref, wfull,
             xbuf, wbuf, acc, bsend, brecv,
             wsend_sems, wrecv_sems, bsend_sems, brecv_sems,
             copy_sems, local_sem):
        me = lax.axis_index("i")
        left = (me - 1) % N_DEV
        right = (me + 1) % N_DEV
        opp = (me + 2) % N_DEV

        barrier_sem = pltpu.get_barrier_semaphore()
        for nbr in [left, right, opp]:
            pl.semaphore_signal(
                barrier_sem, inc=1,
                device_id=(nbr,), device_id_type=pl.DeviceIdType.MESH,
            )
        pl.semaphore_wait(barrier_sem, 3)

        top = pl.ds(0, khalf)
        bot = pl.ds(khalf, khalf)

        def w_rdma(src, dst, dev, si):
            return pltpu.make_async_remote_copy(
                src_ref=src, dst_ref=dst,
                send_sem=wsend_sems.at[si], recv_sem=wrecv_sems.at[si],
                device_id=(dev,), device_id_type=pl.DeviceIdType.MESH)

        cw_a = w_rdma(w_ref.at[top], wfull.at[me, top], right, 0)
        ccw_a = w_rdma(w_ref.at[top], wfull.at[me, top], left, 1)
        cw_a.start()
        ccw_a.start()

        def compute_part(wsrc_ref, k0, ntiles, first):
            def mk_copies(ki, buf):
                cx = pltpu.make_async_copy(
                    x_ref.at[:, pl.ds(ki * KT, KT)],
                    xbuf.at[buf], copy_sems.at[buf])
                cw = pltpu.make_async_copy(
                    wsrc_ref.at[pl.ds(ki * KT, KT), :],
                    wbuf.at[buf], copy_sems.at[2 + buf])
                return cx, cw

            def start_copies(ki, buf):
                cx, cw = mk_copies(ki, buf)
                cx.start()
                cw.start()

            start_copies(k0, 0)

            def step(i, carry):
                ki = k0 + i
                buf = lax.rem(i, 2)

                @pl.when(i + 1 < ntiles)
                def _():
                    start_copies(ki + 1, 1 - buf)

                cx, cw = mk_copies(ki, buf)
                cx.wait()
                cw.wait()
                p = jnp.dot(xbuf[buf].astype(jnp.bfloat16), wbuf[buf],
                            preferred_element_type=jnp.float32)

                if first:
                    @pl.when(i == 0)
                    def _():
                        acc[...] = p

                    @pl.when(i > 0)
                    def _():
                        acc[...] += p
                else:
                    acc[...] += p

                return carry

            lax.fori_loop(0, ntiles, step, 0)

        def finish_gelu():
            y = acc[...]
            c = 0.7978845608028654
            acc[...] = 0.5 * y * (1.0 + jnp.tanh(c * (y + 0.044715 * y * y * y)))

        def compute_block(wsrc_ref):
            compute_part(wsrc_ref, 0, nk, first=True)
            finish_gelu()

        compute_block(w_ref)
        own_cp = pltpu.make_async_copy(
            acc, out_ref.at[pl.ds(me * m_per, m_per), :], local_sem)
        own_cp.start()
        own_cp.wait()

        cw_a.wait_send()
        cw_b = w_rdma(w_ref.at[bot], wfull.at[me, bot], right, 4)
        cw_b.start()
        ccw_a.wait_send()
        ccw_b = w_rdma(w_ref.at[bot], wfull.at[me, bot], left, 5)
        ccw_b.start()

        cw_a.wait_recv()
        ccw_a.wait_recv()

        compute_part(wfull.at[right], 0, nk // 2, first=True)

        cw_b.wait_send()
        fwd_cw = w_rdma(wfull.at[left, top], wfull.at[left, top], right, 2)
        fwd_cw.start()
        ccw_b.wait_send()
        ccw_b.wait_recv()
        fwd_ccw = w_rdma(wfull.at[right, bot], wfull.at[right, bot], left, 3)
        fwd_ccw.start()

        compute_part(wfull.at[right], nk // 2, nk // 2, first=False)
        finish_gelu()
        bsend[0] = acc[...].astype(jnp.bfloat16)
        b_r = pltpu.make_async_remote_copy(
            src_ref=bsend.at[0], dst_ref=brecv.at[0],
            send_sem=bsend_sems.at[0], recv_sem=brecv_sems.at[0],
            device_id=(right,), device_id_type=pl.DeviceIdType.MESH)
        b_r.start()

        cw_b.wait_recv()
        compute_block(wfull.at[left])
        bsend[1] = acc[...].astype(jnp.bfloat16)
        b_l = pltpu.make_async_remote_copy(
            src_ref=bsend.at[1], dst_ref=brecv.at[1],
            send_sem=bsend_sems.at[1], recv_sem=brecv_sems.at[1],
            device_id=(left,), device_id_type=pl.DeviceIdType.MESH)
        b_l.start()

        fwd_cw.wait_recv()
        fwd_ccw.wait_recv()

        compute_part(wfull.at[opp], 0, nk, first=True)
        mhalf = m_per // 2
        c = 0.7978845608028654
        b_o = []
        for h in (0, 1):
            rows = pl.ds(h * mhalf, mhalf)
            y = acc[rows, :]
            g = 0.5 * y * (1.0 + jnp.tanh(c * (y + 0.044715 * y * y * y)))
            bsend[2, rows, :] = g.astype(jnp.bfloat16)
            b = pltpu.make_async_remote_copy(
                src_ref=bsend.at[2, rows, :], dst_ref=brecv.at[2, rows, :],
                send_sem=bsend_sems.at[2 + h], recv_sem=brecv_sems.at[2 + h],
                device_id=(opp,), device_id_type=pl.DeviceIdType.MESH)
            b.start()
            b_o.append(b)

        def store_block(s, origin):
            acc[...] = brecv[s].astype(jnp.float32)
            st = pltpu.make_async_copy(
                acc, out_ref.at[pl.ds(origin * m_per, m_per), :], local_sem)
            st.start()
            st.wait()

        for s, origin in [(0, left), (1, right)]:
            rwait = pltpu.make_async_remote_copy(
                src_ref=bsend.at[s], dst_ref=brecv.at[s],
                send_sem=bsend_sems.at[s], recv_sem=brecv_sems.at[s],
                device_id=(me,), device_id_type=pl.DeviceIdType.MESH)
            rwait.wait_recv()
            store_block(s, origin)
        for h in (0, 1):
            rows = pl.ds(h * mhalf, mhalf)
            rwait = pltpu.make_async_remote_copy(
                src_ref=bsend.at[2, rows, :], dst_ref=brecv.at[2, rows, :],
                send_sem=bsend_sems.at[2 + h], recv_sem=brecv_sems.at[2 + h],
                device_id=(me,), device_id_type=pl.DeviceIdType.MESH)
            rwait.wait_recv()
        store_block(2, opp)

        fwd_cw.wait_send()
        fwd_ccw.wait_send()
        b_r.wait_send()
        b_l.wait_send()
        b_o[0].wait_send()
        b_o[1].wait_send()

    out, _wfull = pl.pallas_call(
        body,
        out_shape=[
            jax.ShapeDtypeStruct((N_DEV * m_per, n_per), jnp.float32),
            jax.ShapeDtypeStruct((N_DEV, k_dim, n_per), jnp.bfloat16),
        ],
        in_specs=[
            pl.BlockSpec(memory_space=pl.ANY),
            pl.BlockSpec(memory_space=pl.ANY),
        ],
        out_specs=[
            pl.BlockSpec(memory_space=pl.ANY),
            pl.BlockSpec(memory_space=pl.ANY),
        ],
        scratch_shapes=[
            pltpu.VMEM((2, m_per, KT), jnp.float32),
            pltpu.VMEM((2, KT, n_per), jnp.bfloat16),
            pltpu.VMEM((m_per, n_per), jnp.float32),
            pltpu.VMEM((3, m_per, n_per), jnp.bfloat16),
            pltpu.VMEM((3, m_per, n_per), jnp.bfloat16),
            pltpu.SemaphoreType.DMA((6,)),
            pltpu.SemaphoreType.DMA((6,)),
            pltpu.SemaphoreType.DMA((4,)),
            pltpu.SemaphoreType.DMA((4,)),
            pltpu.SemaphoreType.DMA((4,)),
            pltpu.SemaphoreType.DMA,
        ],
        compiler_params=pltpu.CompilerParams(
            collective_id=0,
            vmem_limit_bytes=63 * 1024 * 1024,
        ),
    )(x, w_mat.astype(jnp.bfloat16))
    return out
